# baseline (speedup 1.0000x reference)
import functools
import jax
import jax.numpy as jnp
from jax import lax
from jax.experimental import pallas as pl
from jax.experimental.pallas import tpu as pltpu, tpu_sc as plsc

NUM_HIST = 4
PI = float(jnp.pi)

_NB = 2048
_NBP = _NB + 16
_CPW = 60
_NW = 32


def _sc_body(cx_hbm, cy_hbm, feat_hbm, out_hbm, cx_v, cy_v, feat_v, out_v):
    wid = lax.axis_index("s") * 2 + lax.axis_index("c")
    b = wid // 4

    pltpu.sync_copy(cx_hbm.at[wid], cx_v)
    pltpu.sync_copy(cy_hbm.at[wid], cy_v)
    pltpu.sync_copy(feat_hbm.at[b], feat_v)

    inf = jnp.full((16,), jnp.inf, jnp.float32)
    zero16 = jnp.zeros((16,), jnp.float32)
    one = jnp.float32(1.0)
    zero = jnp.float32(0.0)

    cg = [(cx_v[pl.ds(g * 16, 16)], cy_v[pl.ds(g * 16, 16)])
          for g in range(4)]

    def node_body(j, carry):
        f = feat_v[pl.ds(j * 16, 16)]
        px = jnp.full((16,), f[0], jnp.float32)
        py = jnp.full((16,), f[1], jnp.float32)
        ch = jnp.full((16,), f[2], jnp.float32)
        sh = jnp.full((16,), f[3], jnp.float32)
        ln = jnp.full((16,), f[4], jnp.float32)
        st = jnp.full((16,), f[5], jnp.float32)
        ct = jnp.full((16,), f[6], jnp.float32)
        tu = jnp.full((16,), f[7], jnp.float32)

        new = []
        for g in range(4):
            be, ye, bn, pn = carry[g]
            cxg, cyg = cg[g]
            relx = cxg - px
            rely = cyg - py
            x = relx * ch + rely * sh
            y = (-relx) * sh + rely * ch
            y2 = y * y
            big = jnp.full((16,), 1e6, jnp.float32)
            ek = jnp.where(x > 0, jnp.where(x < ln, y2, big), big)
            s = x * x + y2 + jnp.float32(1e-12)

            upd_e = ek < be
            be = jnp.where(upd_e, ek, be)
            ye = jnp.where(upd_e, y, ye)

            crossf = jnp.where((st * x - ct * y) > 0, one, zero)
            upper = y >= 0
            pay_tup = jnp.where(upper, crossf, zero)
            pay_tdn = jnp.where(upper, one, crossf)
            pay = tu * pay_tup + (one - tu) * pay_tdn

            upd_n = s < bn
            bn = jnp.where(upd_n, s, bn)
            pn = jnp.where(upd_n, pay, pn)
            new.append((be, ye, bn, pn))
        return tuple(new)

    init = tuple((inf, zero16, inf, zero16) for _ in range(4))
    final = lax.fori_loop(0, _NB, node_body, init, unroll=4)
    for g in range(4):
        be, ye, bn, pn = final[g]
        coll = jnp.where(be <= bn, jnp.where(ye > 0, one, zero), pn)
        out_v[pl.ds(g * 16, 16)] = coll

    pltpu.sync_copy(out_v, out_hbm.at[wid])


_sc_call = functools.partial(
    pl.kernel,
    out_type=jax.ShapeDtypeStruct((_NW, 64), jnp.float32),
    mesh=plsc.VectorSubcoreMesh(core_axis_name="c", subcore_axis_name="s"),
    scratch_types=[
        pltpu.VMEM((80,), jnp.float32),
        pltpu.VMEM((80,), jnp.float32),
        pltpu.VMEM((_NB * 16,), jnp.float32),
        pltpu.VMEM((64,), jnp.float32),
    ],
)(_sc_body)


def kernel(ptr, agent_batch, infer_position, infer_heading, box, soa_batch,
           soa_position, soa_heading, soa_theta, soa_length):
    ego_index = ptr[:-1]
    pos = infer_position[ego_index, NUM_HIST:]
    yaw = infer_heading[ego_index, NUM_HIST:]
    half = box[ego_index] * 0.5
    signs = jnp.array([[1.0, 1.0], [1.0, -1.0], [-1.0, -1.0], [-1.0, 1.0]],
                      dtype=pos.dtype)
    local = signs[None, :, :] * half[:, None, :]
    c = jnp.cos(yaw)
    s = jnp.sin(yaw)
    lx = local[..., 0][:, None, :]
    ly = local[..., 1][:, None, :]
    gx = pos[..., 0:1] + lx * c[..., None] - ly * s[..., None]
    gy = pos[..., 1:2] + lx * s[..., None] + ly * c[..., None]
    corners = jnp.stack([gx, gy], axis=-1).reshape(-1, 2)

    B = ptr.shape[0] - 1
    T = infer_position.shape[1] - NUM_HIST

    cpad = jnp.zeros((_NW, 20), jnp.float32)
    cx = jnp.concatenate([corners[:, 0].reshape(_NW, _CPW), cpad], axis=1)
    cy = jnp.concatenate([corners[:, 1].reshape(_NW, _CPW), cpad], axis=1)

    feats = jnp.stack([
        soa_position[:, 0].reshape(B, _NB),
        soa_position[:, 1].reshape(B, _NB),
        jnp.cos(soa_heading).reshape(B, _NB),
        jnp.sin(soa_heading).reshape(B, _NB),
        soa_length.reshape(B, _NB),
        jnp.sin(soa_theta).reshape(B, _NB),
        jnp.cos(soa_theta).reshape(B, _NB),
        jnp.where(soa_theta <= PI, 1.0, 0.0).reshape(B, _NB),
    ], axis=-1).astype(jnp.float32)
    feats = jnp.concatenate(
        [feats, jnp.zeros((B, _NB, 8), jnp.float32)], axis=-1)
    feats = feats.reshape(B, _NB * 16)

    out = _sc_call(cx, cy, feats)
    coll = out[:, :_CPW].reshape(-1) > 0
    done = coll.reshape(B, T, 4).any(axis=-1)
    reward = (~coll.reshape(B, T * 4).any(axis=-1)).astype(jnp.float32)
    return done, reward


# speedup vs baseline: 3.1403x; 3.1403x over previous
import functools
import jax
import jax.numpy as jnp
from jax import lax
from jax.experimental import pallas as pl
from jax.experimental.pallas import tpu as pltpu, tpu_sc as plsc

NUM_HIST = 4
PI = float(jnp.pi)

_NB = 2048
_CPW = 60
_NW = 32


def _sc_body(cxy_hbm, feat_hbm, out_hbm, cxy_v, feat_v, out_v):
    wid = lax.axis_index("s") * 2 + lax.axis_index("c")
    b = wid // 4

    pltpu.sync_copy(cxy_hbm.at[wid], cxy_v)
    pltpu.sync_copy(feat_hbm.at[b], feat_v)

    inf = jnp.full((16,), jnp.inf, jnp.float32)
    zero16 = jnp.zeros((16,), jnp.float32)
    one = jnp.float32(1.0)
    zero = jnp.float32(0.0)

    cg = [(cxy_v[pl.ds(g * 16, 16)], cxy_v[pl.ds(80 + g * 16, 16)])
          for g in range(4)]

    def one_node(carry, px, py, ch, sh, ln, st, ct, tun):
        new = []
        for g in range(4):
            be, ye, bn, pn = carry[g]
            cxg, cyg = cg[g]
            relx = cxg - px
            rely = cyg - py
            x = relx * ch + rely * sh
            y = rely * ch - relx * sh
            y2 = y * y
            ek = jnp.where((x > 0) & (x < ln), y2,
                           jnp.full((16,), 1e6, jnp.float32))
            s = x * x + y2 + jnp.float32(1e-12)

            upd_e = ek < be
            be = jnp.where(upd_e, ek, be)
            ye = jnp.where(upd_e, y, ye)

            crossf = jnp.where((st * x - ct * y) > 0, one, zero)
            upper = y >= 0
            pay = jnp.where(upper, jnp.maximum(crossf, tun),
                            jnp.minimum(crossf, tun))

            upd_n = s < bn
            bn = jnp.where(upd_n, s, bn)
            pn = jnp.where(upd_n, pay, pn)
            new.append((be, ye, bn, pn))
        return tuple(new)

    def chunk_body(m, carry):
        base = m * 16
        vpx = feat_v[pl.ds(base, 16)]
        vpy = feat_v[pl.ds(base + _NB, 16)]
        vch = feat_v[pl.ds(base + 2 * _NB, 16)]
        vsh = feat_v[pl.ds(base + 3 * _NB, 16)]
        vln = feat_v[pl.ds(base + 4 * _NB, 16)]
        vst = feat_v[pl.ds(base + 5 * _NB, 16)]
        vct = feat_v[pl.ds(base + 6 * _NB, 16)]
        vtu = feat_v[pl.ds(base + 7 * _NB, 16)]
        for t in range(16):
            carry = one_node(carry,
                             jnp.full((16,), vpx[t], jnp.float32),
                             jnp.full((16,), vpy[t], jnp.float32),
                             jnp.full((16,), vch[t], jnp.float32),
                             jnp.full((16,), vsh[t], jnp.float32),
                             jnp.full((16,), vln[t], jnp.float32),
                             jnp.full((16,), vst[t], jnp.float32),
                             jnp.full((16,), vct[t], jnp.float32),
                             jnp.full((16,), vtu[t], jnp.float32))
        return carry

    init = tuple((inf, zero16, inf, zero16) for _ in range(4))
    final = lax.fori_loop(0, _NB // 16, chunk_body, init)
    for g in range(4):
        be, ye, bn, pn = final[g]
        coll = jnp.where(be <= bn, jnp.where(ye > 0, one, zero), pn)
        out_v[pl.ds(g * 16, 16)] = coll

    pltpu.sync_copy(out_v, out_hbm.at[wid])


_sc_call = functools.partial(
    pl.kernel,
    out_type=jax.ShapeDtypeStruct((_NW, 64), jnp.float32),
    mesh=plsc.VectorSubcoreMesh(core_axis_name="c", subcore_axis_name="s"),
    scratch_types=[
        pltpu.VMEM((160,), jnp.float32),
        pltpu.VMEM((_NB * 8,), jnp.float32),
        pltpu.VMEM((64,), jnp.float32),
    ],
)(_sc_body)


def kernel(ptr, agent_batch, infer_position, infer_heading, box, soa_batch,
           soa_position, soa_heading, soa_theta, soa_length):
    ego_index = ptr[:-1]
    pos = infer_position[ego_index, NUM_HIST:]
    yaw = infer_heading[ego_index, NUM_HIST:]
    half = box[ego_index] * 0.5
    signs = jnp.array([[1.0, 1.0], [1.0, -1.0], [-1.0, -1.0], [-1.0, 1.0]],
                      dtype=pos.dtype)
    local = signs[None, :, :] * half[:, None, :]
    c = jnp.cos(yaw)
    s = jnp.sin(yaw)
    lx = local[..., 0][:, None, :]
    ly = local[..., 1][:, None, :]
    gx = pos[..., 0:1] + lx * c[..., None] - ly * s[..., None]
    gy = pos[..., 1:2] + lx * s[..., None] + ly * c[..., None]
    corners = jnp.stack([gx, gy], axis=-1).reshape(-1, 2)

    B = ptr.shape[0] - 1
    T = infer_position.shape[1] - NUM_HIST

    cpad = jnp.zeros((_NW, 20), jnp.float32)
    cxy = jnp.concatenate([corners[:, 0].reshape(_NW, _CPW), cpad,
                           corners[:, 1].reshape(_NW, _CPW), cpad], axis=1)

    feats = jnp.concatenate([
        soa_position[:, 0].reshape(B, _NB),
        soa_position[:, 1].reshape(B, _NB),
        jnp.cos(soa_heading).reshape(B, _NB),
        jnp.sin(soa_heading).reshape(B, _NB),
        soa_length.reshape(B, _NB),
        jnp.sin(soa_theta).reshape(B, _NB),
        jnp.cos(soa_theta).reshape(B, _NB),
        jnp.where(soa_theta > PI, jnp.float32(1.0),
                  jnp.float32(0.0)).reshape(B, _NB),
    ], axis=1)

    out = _sc_call(cxy, feats)
    coll = out[:, :_CPW].reshape(-1) > 0
    done = coll.reshape(B, T, 4).any(axis=-1)
    reward = (~coll.reshape(B, T * 4).any(axis=-1)).astype(jnp.float32)
    return done, reward
